# Initial kernel scaffold; baseline (speedup 1.0000x reference)
import jax
import jax.numpy as jnp
from jax.experimental import pallas as pl


def kernel(pos, pe):
    raise NotImplementedError("write your pallas kernel here")


# speedup vs baseline: 1.2766x; 1.2766x over previous
import functools

import jax
import jax.numpy as jnp
from jax import lax
from jax.experimental import pallas as pl
from jax.experimental.pallas import tpu as pltpu
from jax.experimental.pallas import tpu_sc as plsc

D_MODEL = 2048
N_ROWS = 4 * 8192
NUM_CORES = 2
NUM_SUBCORES = 16
NW = NUM_CORES * NUM_SUBCORES
RPW = N_ROWS // NW
K = 32
CHUNKS = RPW // K


@functools.partial(
    pl.kernel,
    out_type=jax.ShapeDtypeStruct((N_ROWS, D_MODEL), jnp.float32),
    mesh=plsc.VectorSubcoreMesh(core_axis_name="c", subcore_axis_name="s"),
    scratch_types=[
        pltpu.VMEM((RPW,), jnp.int32),
        pltpu.VMEM((K, D_MODEL), jnp.float32),
        pltpu.SemaphoreType.DMA,
    ],
)
def _sc_gather(pos_hbm, pe_hbm, out_hbm, idx_v, rows_v, sem):
    wid = lax.axis_index("s") * NUM_CORES + lax.axis_index("c")
    base = wid * RPW
    pltpu.sync_copy(pos_hbm.at[pl.ds(base, RPW)], idx_v)

    def chunk_body(i, carry):
        idx_chunk = idx_v.at[pl.ds(i * K, K)]
        pltpu.async_copy(pe_hbm.at[idx_chunk], rows_v, sem).wait()
        pltpu.sync_copy(rows_v, out_hbm.at[pl.ds(base + i * K, K), :])
        return carry

    lax.fori_loop(0, CHUNKS, chunk_body, 0)


def kernel(pos, pe):
    p = pos.reshape(N_ROWS)
    out = _sc_gather(p, pe)
    return out.reshape(pos.shape[0], pos.shape[1], 1, D_MODEL)


# speedup vs baseline: 1.3178x; 1.0323x over previous
import functools

import jax
import jax.numpy as jnp
from jax import lax
from jax.experimental import pallas as pl
from jax.experimental.pallas import tpu as pltpu
from jax.experimental.pallas import tpu_sc as plsc

D_MODEL = 2048
N_ROWS = 4 * 8192
NUM_CORES = 2
NUM_SUBCORES = 16
NW = NUM_CORES * NUM_SUBCORES
RPW = N_ROWS // NW
K = 16
CHUNKS = RPW // K
CHP = CHUNKS // 2


@functools.partial(
    pl.kernel,
    out_type=jax.ShapeDtypeStruct((N_ROWS, D_MODEL), jnp.float32),
    mesh=plsc.VectorSubcoreMesh(core_axis_name="c", subcore_axis_name="s"),
    scratch_types=[
        pltpu.VMEM((RPW,), jnp.int32),
        pltpu.VMEM((K, D_MODEL), jnp.float32),
        pltpu.VMEM((K, D_MODEL), jnp.float32),
        pltpu.SemaphoreType.DMA,
        pltpu.SemaphoreType.DMA,
        pltpu.SemaphoreType.DMA,
    ],
)
def _sc_gather(pos_hbm, pe_hbm, out_hbm, idx_v, buf_a, buf_b, gsem, wsem_a, wsem_b):
    wid = lax.axis_index("s") * NUM_CORES + lax.axis_index("c")
    base = wid * RPW
    pltpu.sync_copy(pos_hbm.at[pl.ds(base, RPW)], idx_v)

    def pair_body(j, carry):
        for buf, wsem, b in ((buf_a, wsem_a, 0), (buf_b, wsem_b, 1)):
            i = 2 * j + b

            @pl.when(j > 0)
            def _():
                pltpu.make_async_copy(buf, out_hbm.at[pl.ds(base, K), :], wsem).wait()

            idx_chunk = idx_v.at[pl.ds(i * K, K)]
            pltpu.async_copy(pe_hbm.at[idx_chunk], buf, gsem).wait()
            pltpu.async_copy(buf, out_hbm.at[pl.ds(base + i * K, K), :], wsem)
        return carry

    lax.fori_loop(0, CHP, pair_body, 0)
    pltpu.make_async_copy(buf_a, out_hbm.at[pl.ds(base, K), :], wsem_a).wait()
    pltpu.make_async_copy(buf_b, out_hbm.at[pl.ds(base, K), :], wsem_b).wait()


def kernel(pos, pe):
    p = pos.reshape(N_ROWS)
    out = _sc_gather(p, pe)
    return out.reshape(pos.shape[0], pos.shape[1], 1, D_MODEL)
